# baseline (speedup 1.0000x reference)
import functools

import jax
import jax.numpy as jnp
from jax import lax
from jax.experimental import pallas as pl
from jax.experimental.pallas import tpu as pltpu
from jax.experimental.pallas import tpu_sc as plsc

MAXP = 100000
EMB = 64
LANES = 16
B = 4096
S = 200
NW = 32
BL = B // NW
DC = EMB // LANES


def _make_kernel():
    info = plsc.get_sparse_core_info()
    assert info.num_cores * info.num_subcores == NW

    mesh = plsc.VectorSubcoreMesh(core_axis_name="c", subcore_axis_name="s")

    @functools.partial(
        pl.kernel,
        mesh=mesh,
        compiler_params=pltpu.CompilerParams(
            use_tc_tiling_on_sc=False, needs_layout_passes=False
        ),
        out_type=jax.ShapeDtypeStruct((S, EMB // 8, NW, 8, BL), jnp.float32),
        scratch_types=[
            pltpu.VMEM((S, BL), jnp.int32),
            pltpu.VMEM((BL,), jnp.int32),
            pltpu.VMEM((BL,), jnp.int32),
            pltpu.VMEM((BL, EMB), jnp.float32),
            pltpu.VMEM((BL, EMB), jnp.float32),
            pltpu.VMEM((EMB // 8, 8, BL), jnp.float32),
            pltpu.VMEM((EMB // 8, 8, BL), jnp.float32),
            pltpu.SemaphoreType.DMA,
            pltpu.SemaphoreType.DMA,
            pltpu.SemaphoreType.DMA,
            pltpu.SemaphoreType.DMA,
            pltpu.SemaphoreType.DMA,
        ],
    )
    def k(x_hbm, tab_hbm, out_hbm, slab, gv0, gv1, r0, r1, t0, t1,
          isem, g0, g1, o0, o1):
        wid = lax.axis_index("s") * info.num_cores + lax.axis_index("c")
        gv = (gv0, gv1)
        rows = (r0, r1)
        tv = (t0, t1)
        gsem = (g0, g1)
        osem = (o0, o1)

        pltpu.async_copy(x_hbm.at[pl.ds(wid * S, S)], slab, isem).wait()

        iota = lax.iota(jnp.int32, LANES)
        f_base = iota * S
        d0c = []
        dsc = []
        for dc in range(DC):
            dvec = iota + dc * LANES
            d0c.append(dvec >> 3)
            dsc.append(dvec & 7)

        def build_gvec(s, g):
            for c in range(BL // LANES):
                f = f_base + (c * LANES * S + s)
                v = plsc.load_gather(slab, [f >> 7, f & 127])
                v = jnp.minimum(jnp.maximum(v, -MAXP), MAXP) + MAXP
                g[pl.ds(c * LANES, LANES)] = v

        def transpose(r, t):
            for b in range(BL):
                bs = jnp.full((LANES,), b, jnp.int32)
                for dc in range(DC):
                    v = r[b, pl.ds(dc * LANES, LANES)]
                    plsc.store_scatter(t, [d0c[dc], dsc[dc], bs], v)

        def drain_gather(p):
            pltpu.make_async_copy(
                tab_hbm.at[pl.ds(0, BL)], rows[p], gsem[p]
            ).wait()

        def store(s, p):
            pltpu.async_copy(tv[p], out_hbm.at[s, :, wid, :, :], osem[p])

        def drain_store(p):
            pltpu.make_async_copy(
                tv[p], out_hbm.at[0, :, wid, :, :], osem[p]
            ).wait()

        def body(s2, carry):
            for p in range(2):
                s = s2 * 2 + p
                np_ = 1 - p

                @pl.when(s >= 2)
                def _():
                    drain_store(p)

                build_gvec(s, gv[p])
                pltpu.async_copy(tab_hbm.at[gv[p]], rows[p], gsem[p])

                @pl.when(s >= 1)
                def _():
                    drain_gather(np_)
                    transpose(rows[np_], tv[np_])
                    store(s - 1, np_)

            return carry

        lax.fori_loop(0, S // 2, body, 0)

        drain_gather(1)
        transpose(rows[1], tv[1])
        store(S - 1, 1)
        drain_store(0)
        drain_store(1)

    return k


def kernel(x, weights):
    assert x.shape == (B, S)
    x2 = x.reshape(B * S // BL, BL)
    o = _make_kernel()(x2, weights)
    return jnp.transpose(o, (2, 4, 0, 1, 3)).reshape(B, S, EMB)


# speedup vs baseline: 2.0355x; 2.0355x over previous
import functools

import jax
import jax.numpy as jnp
from jax import lax
from jax.experimental import pallas as pl
from jax.experimental.pallas import tpu as pltpu
from jax.experimental.pallas import tpu_sc as plsc

MAXP = 100000
EMB = 64
LANES = 16
B = 4096
S = 200
NW = 32
BL = B // NW
DC = EMB // LANES


def _make_kernel():
    info = plsc.get_sparse_core_info()
    assert info.num_cores * info.num_subcores == NW

    mesh = plsc.VectorSubcoreMesh(core_axis_name="c", subcore_axis_name="s")

    @functools.partial(
        pl.kernel,
        mesh=mesh,
        compiler_params=pltpu.CompilerParams(
            use_tc_tiling_on_sc=False, needs_layout_passes=False
        ),
        out_type=jax.ShapeDtypeStruct((S, EMB // 8, NW, 8, BL), jnp.float32),
        scratch_types=[
            pltpu.VMEM((S, BL), jnp.int32),
            pltpu.VMEM((BL,), jnp.int32),
            pltpu.VMEM((BL,), jnp.int32),
            pltpu.VMEM((BL, EMB), jnp.float32),
            pltpu.VMEM((BL, EMB), jnp.float32),
            pltpu.VMEM((EMB // 8, 8, BL), jnp.float32),
            pltpu.VMEM((EMB // 8, 8, BL), jnp.float32),
            pltpu.SemaphoreType.DMA,
            pltpu.SemaphoreType.DMA,
            pltpu.SemaphoreType.DMA,
            pltpu.SemaphoreType.DMA,
            pltpu.SemaphoreType.DMA,
        ],
    )
    def k(xt_hbm, tab_hbm, out_hbm, slab, gv0, gv1, r0, r1, t0, t1,
          isem, g0, g1, o0, o1):
        wid = lax.axis_index("s") * info.num_cores + lax.axis_index("c")
        gv = (gv0, gv1)
        rows = (r0, r1)
        tv = (t0, t1)
        gsem = (g0, g1)
        osem = (o0, o1)

        pltpu.async_copy(
            xt_hbm.at[:, pl.ds(wid * BL, BL)], slab, isem
        ).wait()

        iota = lax.iota(jnp.int32, LANES)
        perms = [(iota + j) & 15 for j in range(LANES)]
        cols = [iota + d16 * LANES for d16 in range(DC)]
        d0s = [(iota + d16 * LANES) >> 3 for d16 in range(DC)]
        dss = iota & 7

        def build_gvec(s, g):
            for c in range(BL // LANES):
                v = slab[s, pl.ds(c * LANES, LANES)]
                v = jnp.minimum(jnp.maximum(v, -MAXP), MAXP) + MAXP
                g[pl.ds(c * LANES, LANES)] = v

        def transpose(r, t):
            def tbody(b16, carry):
                b0 = b16 * LANES
                for d16 in range(DC):
                    for j in range(LANES):
                        ridx = perms[j] + b0
                        v = plsc.load_gather(r, [ridx, cols[d16]])
                        plsc.store_scatter(t, [d0s[d16], dss, ridx], v)
                return carry

            lax.fori_loop(0, BL // LANES, tbody, 0)

        def drain_gather(p):
            pltpu.make_async_copy(
                tab_hbm.at[pl.ds(0, BL)], rows[p], gsem[p]
            ).wait()

        def store(s, p):
            pltpu.async_copy(tv[p], out_hbm.at[s, :, wid, :, :], osem[p])

        def drain_store(p):
            pltpu.make_async_copy(
                tv[p], out_hbm.at[0, :, wid, :, :], osem[p]
            ).wait()

        def body(s2, carry):
            for p in range(2):
                s = s2 * 2 + p
                np_ = 1 - p

                @pl.when(s >= 2)
                def _():
                    drain_store(p)

                build_gvec(s, gv[p])
                pltpu.async_copy(tab_hbm.at[gv[p]], rows[p], gsem[p])

                @pl.when(s >= 1)
                def _():
                    drain_gather(np_)
                    transpose(rows[np_], tv[np_])
                    store(s - 1, np_)

            return carry

        lax.fori_loop(0, S // 2, body, 0)

        drain_gather(1)
        transpose(rows[1], tv[1])
        store(S - 1, 1)
        drain_store(0)
        drain_store(1)

    return k


def kernel(x, weights):
    assert x.shape == (B, S)
    o = _make_kernel()(x.T, weights)
    return jnp.transpose(o, (2, 4, 0, 1, 3)).reshape(B, S, EMB)


# speedup vs baseline: 3.9097x; 1.9208x over previous
import functools

import jax
import jax.numpy as jnp
from jax import lax
from jax.experimental import pallas as pl
from jax.experimental.pallas import tpu as pltpu
from jax.experimental.pallas import tpu_sc as plsc

MAXP = 100000
EMB = 64
LANES = 16
B = 4096
S = 200
NW = 32
BL = B // NW
DC = EMB // LANES


def _make_kernel():
    info = plsc.get_sparse_core_info()
    assert info.num_cores * info.num_subcores == NW

    mesh = plsc.VectorSubcoreMesh(core_axis_name="c", subcore_axis_name="s")

    @functools.partial(
        pl.kernel,
        mesh=mesh,
        compiler_params=pltpu.CompilerParams(
            use_tc_tiling_on_sc=False, needs_layout_passes=False
        ),
        out_type=jax.ShapeDtypeStruct((S, EMB // 8, NW, 8, BL), jnp.float32),
        scratch_types=[
            pltpu.VMEM((S, BL), jnp.int32),
            pltpu.VMEM((BL,), jnp.int32),
            pltpu.VMEM((BL,), jnp.int32),
            pltpu.VMEM((BL, EMB), jnp.float32),
            pltpu.VMEM((BL, EMB), jnp.float32),
            pltpu.VMEM((EMB // 8, 8, BL), jnp.float32),
            pltpu.VMEM((EMB // 8, 8, BL), jnp.float32),
            pltpu.SemaphoreType.DMA,
            pltpu.SemaphoreType.DMA,
            pltpu.SemaphoreType.DMA,
            pltpu.SemaphoreType.DMA,
            pltpu.SemaphoreType.DMA,
        ],
    )
    def k(xt_hbm, tab_hbm, out_hbm, slab, gv0, gv1, r0, r1, t0, t1,
          isem, g0, g1, o0, o1):
        wid = lax.axis_index("s") * info.num_cores + lax.axis_index("c")
        gv = (gv0, gv1)
        rows = (r0, r1)
        tv = (t0, t1)
        gsem = (g0, g1)
        osem = (o0, o1)

        pltpu.async_copy(
            xt_hbm.at[:, pl.ds(wid * BL, BL)], slab, isem
        ).wait()

        iota = lax.iota(jnp.int32, LANES)
        perms = [(iota + j) & 15 for j in range(LANES)]
        cols = [iota + d16 * LANES for d16 in range(DC)]
        d0s = [(iota + d16 * LANES) >> 3 for d16 in range(DC)]
        dss = iota & 7

        def build_gvec(s, g):
            for c in range(BL // LANES):
                v = slab[s, pl.ds(c * LANES, LANES)]
                v = jnp.minimum(jnp.maximum(v, -MAXP), MAXP) + MAXP
                g[pl.ds(c * LANES, LANES)] = v

        def transpose(r, t):
            @functools.partial(plsc.parallel_loop, 0, BL // LANES)
            def _(b16):
                b0 = b16 * LANES
                for d16 in range(DC):
                    for j in range(LANES):
                        ridx = perms[j] + b0
                        v = plsc.load_gather(r, [ridx, cols[d16]])
                        plsc.store_scatter(t, [d0s[d16], dss, ridx], v)

        def drain_gather(p):
            pltpu.make_async_copy(
                tab_hbm.at[pl.ds(0, BL)], rows[p], gsem[p]
            ).wait()

        def store(s, p):
            pltpu.async_copy(tv[p], out_hbm.at[s, :, wid, :, :], osem[p])

        def drain_store(p):
            pltpu.make_async_copy(
                tv[p], out_hbm.at[0, :, wid, :, :], osem[p]
            ).wait()

        def body(s2, carry):
            for p in range(2):
                s = s2 * 2 + p
                np_ = 1 - p

                @pl.when(s >= 2)
                def _():
                    drain_store(p)

                build_gvec(s, gv[p])
                pltpu.async_copy(tab_hbm.at[gv[p]], rows[p], gsem[p])

                @pl.when(s >= 1)
                def _():
                    drain_gather(np_)
                    transpose(rows[np_], tv[np_])
                    store(s - 1, np_)

            return carry

        lax.fori_loop(0, S // 2, body, 0)

        drain_gather(1)
        transpose(rows[1], tv[1])
        store(S - 1, 1)
        drain_store(0)
        drain_store(1)

    return k


def kernel(x, weights):
    assert x.shape == (B, S)
    o = _make_kernel()(x.T, weights)
    return jnp.transpose(o, (2, 4, 0, 1, 3)).reshape(B, S, EMB)
